# Initial kernel scaffold; baseline (speedup 1.0000x reference)
import jax
import jax.numpy as jnp
from jax.experimental import pallas as pl


def kernel(input, weight):
    raise NotImplementedError("write your pallas kernel here")


# speedup vs baseline: 1.5610x; 1.5610x over previous
import functools

import jax
import jax.numpy as jnp
from jax import lax
from jax.experimental import pallas as pl
from jax.experimental.pallas import tpu as pltpu
from jax.experimental.pallas import tpu_sc as plsc

BATCH = 16384
FIELDS = 26
DIM = 32
TOTAL = BATCH * FIELDS

_info = plsc.get_sparse_core_info()
_NC = _info.num_cores
_NS = _info.num_subcores
_NW = _NC * _NS
_B_PER_W = TOTAL // _NW
_NCHUNK = 8
_C = _B_PER_W // _NCHUNK


def _make_kernel():
    mesh = plsc.VectorSubcoreMesh(core_axis_name="c", subcore_axis_name="s")

    @functools.partial(
        pl.kernel,
        mesh=mesh,
        compiler_params=pltpu.CompilerParams(use_tc_tiling_on_sc=False),
        out_type=jax.ShapeDtypeStruct((TOTAL, DIM), jnp.float32),
        scratch_types=[
            pltpu.VMEM((_C,), jnp.int32),
            pltpu.VMEM((_C, DIM), jnp.float32),
            pltpu.SemaphoreType.DMA,
        ],
    )
    def gather_kernel(idx_hbm, table_hbm, out_hbm, idx_v, rows_v, sem):
        wid = lax.axis_index("s") * _NC + lax.axis_index("c")
        base = wid * _B_PER_W

        for j in range(_NCHUNK):
            off = base + j * _C
            pltpu.sync_copy(idx_hbm.at[pl.ds(off, _C)], idx_v)
            pltpu.async_copy(table_hbm.at[idx_v], rows_v, sem).wait()
            pltpu.sync_copy(rows_v, out_hbm.at[pl.ds(off, _C)])

    return gather_kernel


_gather = _make_kernel()


@jax.jit
def kernel(input, weight):
    idx_flat = input.reshape(TOTAL).astype(jnp.int32)
    out = _gather(idx_flat, weight)
    return out.reshape(BATCH, FIELDS, DIM)


# speedup vs baseline: 1.5750x; 1.0090x over previous
import functools

import jax
import jax.numpy as jnp
from jax import lax
from jax.experimental import pallas as pl
from jax.experimental.pallas import tpu as pltpu
from jax.experimental.pallas import tpu_sc as plsc

BATCH = 16384
FIELDS = 26
DIM = 32
TOTAL = BATCH * FIELDS

_info = plsc.get_sparse_core_info()
_NC = _info.num_cores
_NS = _info.num_subcores
_NW = _NC * _NS
_B_PER_W = TOTAL // _NW
_NCHUNK = 8
_C = _B_PER_W // _NCHUNK


def _make_kernel():
    mesh = plsc.VectorSubcoreMesh(core_axis_name="c", subcore_axis_name="s")

    @functools.partial(
        pl.kernel,
        mesh=mesh,
        compiler_params=pltpu.CompilerParams(use_tc_tiling_on_sc=False),
        out_type=jax.ShapeDtypeStruct((TOTAL, DIM), jnp.float32),
        scratch_types=[
            pltpu.VMEM((_NCHUNK, _C), jnp.int32),
            pltpu.VMEM((2, _C, DIM), jnp.float32),
            pltpu.SemaphoreType.DMA,
            pltpu.SemaphoreType.DMA,
            pltpu.SemaphoreType.DMA,
            pltpu.SemaphoreType.DMA,
        ],
    )
    def gather_kernel(idx_hbm, table_hbm, out_hbm, idx_v, rows_v,
                      gsem0, gsem1, osem0, osem1):
        wid = lax.axis_index("s") * _NC + lax.axis_index("c")
        base = wid * _B_PER_W
        gsems = (gsem0, gsem1)
        osems = (osem0, osem1)

        pltpu.sync_copy(idx_hbm.at[wid], idx_v)

        g = [
            pltpu.async_copy(table_hbm.at[idx_v.at[b]], rows_v.at[b], gsems[b])
            for b in range(2)
        ]
        o = [None, None]
        for j in range(_NCHUNK):
            slot = j % 2
            g[slot].wait()
            o[slot] = pltpu.async_copy(
                rows_v.at[slot], out_hbm.at[pl.ds(base + j * _C, _C)],
                osems[slot],
            )
            if j + 2 < _NCHUNK:
                o[slot].wait()
                g[slot] = pltpu.async_copy(
                    table_hbm.at[idx_v.at[j + 2]], rows_v.at[slot],
                    gsems[slot],
                )
        o[0].wait()
        o[1].wait()

    return gather_kernel


_gather = _make_kernel()


@jax.jit
def kernel(input, weight):
    idx_flat = input.reshape(_NW, _NCHUNK, _C).astype(jnp.int32)
    out = _gather(idx_flat, weight)
    return out.reshape(BATCH, FIELDS, DIM)
